# baseline (speedup 1.0000x reference)
import jax
import jax.numpy as jnp
from jax import lax
from jax.experimental import pallas as pl
from jax.experimental.pallas import tpu as pltpu
from jax.experimental.pallas import tpu_sc as plsc

_NC = 2
_NS = 16
_NW = _NC * _NS
_ROWS_PER_W = 1024 // _NW


def _sc_argmax(attn_hbm, idx_hbm, slab_v, idx_v):
    wid = lax.axis_index("s") * _NC + lax.axis_index("c")
    base = wid * _ROWS_PER_W
    b = base // 256
    l0 = base % 256
    pltpu.sync_copy(attn_hbm.at[b, :, pl.ds(l0, _ROWS_PER_W), :], slab_v)

    lane = lax.iota(jnp.int32, 16)
    perms = [lane ^ sh for sh in (1, 2, 4, 8)]

    def allmax(x):
        for p in perms:
            x = jnp.maximum(x, jnp.take(x, p))
        return x

    def allmin(x):
        for p in perms:
            x = jnp.minimum(x, jnp.take(x, p))
        return x

    def grp_body(g, _):
        accv = jnp.zeros((16,), jnp.int32)
        for rr in range(16):
            r = g * 16 + rr
            mval = jnp.full((16,), -jnp.inf, jnp.float32)
            midx = jnp.full((16,), 9999, jnp.int32)
            for c in range(16):
                v = slab_v[0, r, pl.ds(c * 16, 16)]
                for h in range(1, 8):
                    v = v + slab_v[h, r, pl.ds(c * 16, 16)]
                better = v > mval
                mval = jnp.where(better, v, mval)
                midx = jnp.where(better, c * 16 + lane, midx)
            gm = allmax(mval)
            best = allmin(jnp.where(mval == gm, midx, 9999))
            accv = jnp.where(lane == rr, best, accv)
        idx_v[pl.ds(g * 16, 16)] = accv
        return 0

    lax.fori_loop(0, _ROWS_PER_W // 16, grp_body, 0)
    pltpu.sync_copy(idx_v, idx_hbm.at[pl.ds(base, _ROWS_PER_W)])


def _tc_kernel(idx_ref, c_ref, out_ref):
    idx = idx_ref[0]
    col = lax.broadcasted_iota(jnp.int32, (256, 256), 1)
    sel = (col == idx).astype(jnp.float32)

    n = lax.broadcasted_iota(jnp.int32, (4096, 256), 0)
    l = lax.broadcasted_iota(jnp.int32, (4096, 256), 1)
    s = (((n // 256) * 16 + (n % 64) // 4) == l).astype(jnp.float32)

    c2 = c_ref[0]
    pooled = lax.dot_general(
        c2, s, (((1,), (0,)), ((), ())), preferred_element_type=jnp.float32)
    g = lax.dot_general(
        pooled, sel, (((1,), (1,)), ((), ())),
        preferred_element_type=jnp.float32)
    out = lax.dot_general(
        g, s, (((1,), (1,)), ((), ())), preferred_element_type=jnp.float32)
    out_ref[0] = out * 0.0625


def kernel(A, B, C, D, attn):
    Bn, Cc, H, W = C.shape
    N = H * W

    mesh = plsc.VectorSubcoreMesh(core_axis_name="c", subcore_axis_name="s")
    idx_flat = pl.kernel(
        _sc_argmax,
        mesh=mesh,
        out_type=jax.ShapeDtypeStruct((Bn * 256,), jnp.int32),
        scratch_types=[
            pltpu.VMEM((8, _ROWS_PER_W, 256), jnp.float32),
            pltpu.VMEM((_ROWS_PER_W,), jnp.int32),
        ],
    )(attn)

    c2 = C.reshape(Bn, Cc, N)
    out = pl.pallas_call(
        _tc_kernel,
        grid=(Bn,),
        in_specs=[
            pl.BlockSpec((1, 256, 1), lambda bb: (bb, 0, 0)),
            pl.BlockSpec((1, Cc, N), lambda bb: (bb, 0, 0)),
        ],
        out_specs=pl.BlockSpec((1, Cc, N), lambda bb: (bb, 0, 0)),
        out_shape=jax.ShapeDtypeStruct((Bn, Cc, N), jnp.float32),
    )(idx_flat.reshape(Bn, 256, 1), c2)
    return out.reshape(Bn, Cc, H, W)


# speedup vs baseline: 1.0835x; 1.0835x over previous
import jax
import jax.numpy as jnp
from jax import lax
from jax.experimental import pallas as pl
from jax.experimental.pallas import tpu as pltpu
from jax.experimental.pallas import tpu_sc as plsc

_NC = 2
_NS = 16
_NW = _NC * _NS


def _tc_kernel(attn_ref, c_ref, idx_ref, pooled_ref):
    coarse = attn_ref[0, 0]
    for h in range(1, 8):
        coarse = coarse + attn_ref[0, h]
    coarse = coarse * 0.125

    idx_ref[0] = jnp.argmax(coarse, axis=1, keepdims=True)

    n = lax.broadcasted_iota(jnp.int32, (4096, 256), 0)
    l = lax.broadcasted_iota(jnp.int32, (4096, 256), 1)
    s = (((n // 256) * 16 + (n % 64) // 4) == l).astype(jnp.float32)
    pooled = lax.dot_general(
        c_ref[0], s, (((1,), (0,)), ((), ())),
        preferred_element_type=jnp.float32)
    pooled_ref[0] = pooled * 0.0625


def _sc_expand(pooled_hbm, idx_hbm, out_hbm, rows_v, idx_v, out_v):
    wid = lax.axis_index("s") * _NC + lax.axis_index("c")
    b = wid // 8
    ch0 = (wid % 8) * 24
    pltpu.sync_copy(pooled_hbm.at[b, pl.ds(ch0 * 256, 24 * 256)], rows_v)
    pltpu.sync_copy(idx_hbm.at[pl.ds(b * 256, 256)], idx_v)

    lane = lax.iota(jnp.int32, 16)
    expand_perms = [4 * q + lane // 4 for q in range(4)]
    idx_chunks = [idx_v[pl.ds(c * 16, 16)] for c in range(16)]

    def row_body(r, rbase):
        for c in range(16):
            g = plsc.load_gather(rows_v, [rbase + idx_chunks[c]])
            for q in range(4):
                t = jnp.take(g, expand_perms[q])
                for j in range(4):
                    out_v[r, pl.ds(256 * c + 64 * j + 16 * q, 16)] = t
        return rbase + 256

    lax.fori_loop(0, 24, row_body, jnp.zeros((16,), jnp.int32))
    pltpu.sync_copy(out_v, out_hbm.at[b, pl.ds(ch0, 24), :])


def kernel(A, B, C, D, attn):
    Bn, Cc, H, W = C.shape
    N = H * W
    c2 = C.reshape(Bn, Cc, N)

    idx, pooled = pl.pallas_call(
        _tc_kernel,
        grid=(Bn,),
        in_specs=[
            pl.BlockSpec((1, 8, 256, 256), lambda bb: (bb, 0, 0, 0)),
            pl.BlockSpec((1, Cc, N), lambda bb: (bb, 0, 0)),
        ],
        out_specs=[
            pl.BlockSpec((1, 256, 1), lambda bb: (bb, 0, 0)),
            pl.BlockSpec((1, Cc, 256), lambda bb: (bb, 0, 0)),
        ],
        out_shape=[
            jax.ShapeDtypeStruct((Bn, 256, 1), jnp.int32),
            jax.ShapeDtypeStruct((Bn, Cc, 256), jnp.float32),
        ],
    )(attn, c2)

    mesh = plsc.VectorSubcoreMesh(core_axis_name="c", subcore_axis_name="s")
    out = pl.kernel(
        _sc_expand,
        mesh=mesh,
        compiler_params=pltpu.CompilerParams(needs_layout_passes=False),
        out_type=jax.ShapeDtypeStruct((Bn, Cc, N), jnp.float32),
        scratch_types=[
            pltpu.VMEM((24 * 256,), jnp.float32),
            pltpu.VMEM((256,), jnp.int32),
            pltpu.VMEM((24, N), jnp.float32),
        ],
    )(pooled.reshape(Bn, Cc * 256), idx.reshape(Bn * 256))
    return out.reshape(Bn, Cc, H, W)
